# baseline (speedup 1.0000x reference)
import functools

import jax
import jax.numpy as jnp
from jax import lax
from jax.experimental import pallas as pl
from jax.experimental.pallas import tpu as pltpu
from jax.experimental.pallas import tpu_sc as plsc

_BATCH = 16384
_HIST = 20
_B = _BATCH * _HIST
_D = 32
_DW = _D // 2
_CHUNK = 128
_NBUF = 4
_NC = 2
_NS = 16


@functools.cache
def _build():
    nw = _NC * _NS
    b_per_w = _B // nw
    n_chunks = b_per_w // _CHUNK
    mesh = plsc.VectorSubcoreMesh(
        core_axis_name="c", subcore_axis_name="s",
        num_cores=_NC, num_subcores=_NS,
    )

    @functools.partial(
        pl.kernel,
        out_type=jax.ShapeDtypeStruct((_B, _DW), jnp.int32),
        mesh=mesh,
        scratch_types=[
            pltpu.VMEM((n_chunks, _CHUNK), jnp.int32),
            pltpu.VMEM((_NBUF, _CHUNK, _DW), jnp.int32),
        ] + [pltpu.SemaphoreType.DMA] * _NBUF,
        compiler_params=pltpu.CompilerParams(use_tc_tiling_on_sc=False),
    )
    def embed(idx_hbm, table_hbm, out_hbm, idx_v, rows_v, *sems):
        wid = lax.axis_index("s") * _NC + lax.axis_index("c")
        chunk0 = wid * n_chunks
        row0 = wid * b_per_w

        pltpu.sync_copy(idx_hbm.at[pl.ds(chunk0, n_chunks)], idx_v)

        def issue(j, b):
            pltpu.async_copy(table_hbm.at[idx_v.at[j]], rows_v.at[b], sems[b])

        def wait(j, b):
            pltpu.make_async_copy(
                table_hbm.at[idx_v.at[j]], rows_v.at[b], sems[b]
            ).wait()

        def store(j, b):
            pltpu.sync_copy(
                rows_v.at[b], out_hbm.at[pl.ds(row0 + j * _CHUNK, _CHUNK)]
            )

        for b in range(_NBUF):
            issue(b, b)

        @pl.loop(0, n_chunks - _NBUF, step=_NBUF)
        def _(g):
            for b in range(_NBUF):
                j = g + b
                wait(j, b)
                store(j, b)
                issue(j + _NBUF, b)

        for b in range(_NBUF):
            j = n_chunks - _NBUF + b
            wait(j, b)
            store(j, b)

    return embed


def kernel(inputs, embedding):
    idx = inputs.reshape(_B // _CHUNK, _CHUNK)
    table_i32 = lax.bitcast_convert_type(
        embedding.reshape(embedding.shape[0], _DW, 2), jnp.int32
    )
    out = _build()(idx, table_i32)
    out_bf16 = lax.bitcast_convert_type(out, jnp.bfloat16)
    return out_bf16.reshape(_BATCH, _HIST, _D)
